# baseline (speedup 1.0000x reference)
import functools

import numpy as np

import jax
import jax.numpy as jnp
from jax import lax
from jax.experimental import pallas as pl
from jax.experimental.pallas import tpu as pltpu
from jax.experimental.pallas import tpu_sc as plsc

NC = 2
NS = 16
LANES = 16


@functools.partial(jax.jit, static_argnums=(3, 4, 5, 6))
def _cp_partials(coords_t, table_t, weights, B, H, V, R):
    assert R == NC * NS
    QB = 4096
    NQ = B // QB
    mesh = plsc.VectorSubcoreMesh(core_axis_name="c", subcore_axis_name="s")

    @functools.partial(
        pl.kernel,
        out_type=jax.ShapeDtypeStruct((R, B), jnp.float32),
        mesh=mesh,
        scratch_types=[
            pltpu.VMEM((V,), jnp.float32),
            pltpu.VMEM((B,), jnp.float32),
            pltpu.VMEM((QB,), jnp.int32),
            pltpu.VMEM((R,), jnp.float32),
        ],
        compiler_params=pltpu.CompilerParams(needs_layout_passes=False),
    )
    def k(ct_hbm, tab_hbm, w_hbm, p_hbm,
          row_v, prod_v, cq_v, w_v):
        def i32(x):
            return lax.convert_element_type(x, jnp.int32)

        c = i32(lax.axis_index("c"))
        s = i32(lax.axis_index("s"))
        r = c * NS + s
        pltpu.sync_copy(w_hbm, w_v)
        w_bc = plsc.load_gather(w_v, [jnp.full((LANES,), r, jnp.int32)])

        def gather_pass(m, first):
            pltpu.sync_copy(tab_hbm.at[m, r], row_v)
            for q in range(NQ):
                pltpu.sync_copy(ct_hbm.at[m, pl.ds(np.int32(q * QB), QB)],
                                cq_v)

                @pl.loop(np.int32(0), np.int32(QB // LANES), unroll=8)
                def _(iv):
                    iv = i32(iv)
                    off = iv * LANES
                    idx = cq_v[pl.ds(off, LANES)]
                    vals = plsc.load_gather(row_v, [idx])
                    pslice = pl.ds(np.int32(q * QB) + off, LANES)
                    if first:
                        prod_v[pslice] = vals * w_bc
                    else:
                        prod_v[pslice] = prod_v[pslice] * vals

        gather_pass(np.int32(0), True)

        @pl.loop(np.int32(1), np.int32(H))
        def _(m):
            gather_pass(i32(m), False)

        pltpu.sync_copy(prod_v, p_hbm.at[r])

    return k(coords_t, table_t, weights)


def _combine(p):
    def k2(p_ref, o_ref):
        o_ref[...] = jnp.sum(p_ref[...], axis=0)

    return pl.pallas_call(
        k2,
        out_shape=jax.ShapeDtypeStruct((p.shape[1],), jnp.float32),
    )(p)


def kernel(coords, factors, weights):
    H, V, R = factors.shape
    B = coords.shape[0]
    coords_t = coords.astype(jnp.int32).T
    table_t = jnp.transpose(factors, (0, 2, 1))
    with jax.enable_x64(False):
        p = _cp_partials(coords_t, table_t, weights.astype(jnp.float32),
                         B, H, V, R)
        return _combine(p)


# speedup vs baseline: 1.5192x; 1.5192x over previous
import functools

import numpy as np

import jax
import jax.numpy as jnp
from jax import lax
from jax.experimental import pallas as pl
from jax.experimental.pallas import tpu as pltpu
from jax.experimental.pallas import tpu_sc as plsc

NC = 2
NS = 16
LANES = 16


@functools.partial(jax.jit, static_argnums=(3, 4, 5, 6))
def _cp_partials(coords_t, table_t, weights, B, H, V, R):
    assert R == NC * NS
    QB = 4096
    NQ = B // QB
    mesh = plsc.VectorSubcoreMesh(core_axis_name="c", subcore_axis_name="s")

    GC = 128
    NG = QB // GC

    @functools.partial(
        pl.kernel,
        out_type=jax.ShapeDtypeStruct((R, B), jnp.float32),
        mesh=mesh,
        scratch_types=[
            pltpu.VMEM_SHARED((V,), jnp.float32),
            pltpu.VMEM((B,), jnp.float32),
            pltpu.VMEM((QB,), jnp.int32),
            pltpu.VMEM((QB,), jnp.int32),
            pltpu.VMEM((QB,), jnp.float32),
            pltpu.VMEM((QB,), jnp.float32),
            pltpu.VMEM((R,), jnp.float32),
            pltpu.SemaphoreType.DMA,
            pltpu.SemaphoreType.DMA,
        ],
        compiler_params=pltpu.CompilerParams(needs_layout_passes=False),
    )
    def k(ct_hbm, tab_hbm, w_hbm, p_hbm,
          row_s, prod_v, cq_a, cq_b, val_a, val_b, w_v, sem_a, sem_b):
        def i32(x):
            return lax.convert_element_type(x, jnp.int32)

        c = i32(lax.axis_index("c"))
        s = i32(lax.axis_index("s"))
        r = c * NS + s
        cqs = (cq_a, cq_b)
        vals_bufs = (val_a, val_b)
        sems = (sem_a, sem_b)
        pltpu.sync_copy(w_hbm, w_v)
        w_bc = plsc.load_gather(w_v, [jnp.full((LANES,), r, jnp.int32)])

        def fire(par):
            for j in range(NG):
                sl = pl.ds(np.int32(j * GC), GC)
                pltpu.async_copy(row_s.at[cqs[par].at[sl]],
                                 vals_bufs[par].at[sl], sems[par])

        def drain(par):
            for j in range(NG):
                sl = pl.ds(np.int32(j * GC), GC)
                pltpu.make_async_copy(row_s.at[cqs[par].at[sl]],
                                      vals_bufs[par].at[sl],
                                      sems[par]).wait()

        def mul_pass(q, par, first):
            vb = vals_bufs[par]

            @pl.loop(np.int32(0), np.int32(QB // LANES), unroll=8)
            def _(iv):
                off = i32(iv) * LANES
                vals = vb[pl.ds(off, LANES)]
                ps = pl.ds(np.int32(q * QB) + off, LANES)
                if first:
                    prod_v[ps] = vals * w_bc
                else:
                    prod_v[ps] = prod_v[ps] * vals

        def gather_pass(m, first):
            pltpu.sync_copy(tab_hbm.at[m, r], row_s)
            pltpu.sync_copy(ct_hbm.at[m, pl.ds(np.int32(0), QB)], cqs[0])
            fire(0)
            for q in range(NQ):
                par = q % 2
                if q + 1 < NQ:
                    nxt = (q + 1) % 2
                    pltpu.sync_copy(
                        ct_hbm.at[m, pl.ds(np.int32((q + 1) * QB), QB)],
                        cqs[nxt])
                    drain(par)
                    fire(nxt)
                else:
                    drain(par)
                mul_pass(q, par, first)

        gather_pass(np.int32(0), True)

        @pl.loop(np.int32(1), np.int32(H))
        def _(m):
            gather_pass(i32(m), False)

        pltpu.sync_copy(prod_v, p_hbm.at[r])

    return k(coords_t, table_t, weights)


def _combine(p):
    def k2(p_ref, o_ref):
        o_ref[...] = jnp.sum(p_ref[...], axis=0)

    return pl.pallas_call(
        k2,
        out_shape=jax.ShapeDtypeStruct((p.shape[1],), jnp.float32),
    )(p)


def kernel(coords, factors, weights):
    H, V, R = factors.shape
    B = coords.shape[0]
    coords_t = coords.astype(jnp.int32).T
    table_t = jnp.transpose(factors, (0, 2, 1))
    with jax.enable_x64(False):
        p = _cp_partials(coords_t, table_t, weights.astype(jnp.float32),
                         B, H, V, R)
        return _combine(p)
